# baseline (speedup 1.0000x reference)
import functools

import jax
import jax.numpy as jnp
from jax import lax
from jax.experimental import pallas as pl
from jax.experimental.pallas import tpu as pltpu
from jax.experimental.pallas import tpu_sc as plsc

_LANES = 16
_CHUNK = 64
_NBUF = 8
_LOOK = 4


@functools.cache
def _build(B, T, V, D):
    info = plsc.get_sparse_core_info()
    NC, NS = info.num_cores, info.num_subcores
    NW = NC * NS
    FLAT = B * T
    assert T % NW == 0 and D % _LANES == 0
    P = T // NW
    G = _CHUNK // P
    assert B % G == 0
    n_chunks = B // G
    assert n_chunks % _NBUF == 0 and n_chunks >= 2 * _NBUF
    mesh = plsc.VectorSubcoreMesh(core_axis_name="c", subcore_axis_name="s")

    @functools.partial(
        pl.kernel,
        mesh=mesh,
        out_type=jax.ShapeDtypeStruct((FLAT, D), jnp.float32),
        scratch_types=(
            [
                pltpu.VMEM((n_chunks, _CHUNK), jnp.int32),
                pltpu.VMEM((P, D), jnp.float32),
            ]
            + [pltpu.VMEM((_CHUNK, D), jnp.float32) for _ in range(_NBUF)]
            + [pltpu.SemaphoreType.DMA for _ in range(2 * _NBUF)]
        ),
    )
    def k(tok_hbm, table_hbm, pos_hbm, out_hbm, idx_v, pos_v, *bufs_and_sems):
        rows = bufs_and_sems[:_NBUF]
        gsem = bufs_and_sems[_NBUF:2 * _NBUF]
        ssem = bufs_and_sems[2 * _NBUF:]
        wid = lax.axis_index("s") * NC + lax.axis_index("c")
        pltpu.sync_copy(tok_hbm.at[wid], idx_v)
        pltpu.sync_copy(pos_hbm.at[pl.ds(wid * P, P)], pos_v)

        def start_gather(c, b):
            pltpu.async_copy(table_hbm.at[idx_v.at[c]], rows[b], gsem[b])

        def wait_gather(c, b):
            pltpu.make_async_copy(
                table_hbm.at[idx_v.at[c]], rows[b], gsem[b]).wait()

        def add_pos(rows_v, c):
            @plsc.parallel_loop(0, P, 1, unroll=2)
            def add_p(p):
                for j in range(D // _LANES):
                    sl = pl.ds(j * _LANES, _LANES)
                    pv = pos_v[p, sl]
                    for g in range(G):
                        r = g * P + p
                        rows_v[r, sl] = rows_v[r, sl] + pv

        def store_chunk(rows_v, c, sem):
            for g in range(G):
                row0 = (c * G + g) * T + wid * P
                pltpu.async_copy(
                    rows_v.at[pl.ds(g * P, P)],
                    out_hbm.at[pl.ds(row0, P)], sem)

        def wait_store(rows_v, sem):
            pltpu.make_async_copy(
                rows_v, out_hbm.at[pl.ds(0, _CHUNK)], sem).wait()

        niter = n_chunks // _NBUF
        for c0 in range(_LOOK):
            start_gather(c0, c0)

        def loop_body(i, carry):
            for b0 in range(_NBUF):
                c = _NBUF * i + b0
                wait_gather(c, b0)
                fb = (b0 + _LOOK) % _NBUF
                if b0 >= _NBUF - _LOOK:
                    wait_store(rows[fb], ssem[fb])
                else:
                    @pl.when(i > 0)
                    def _():
                        wait_store(rows[fb], ssem[fb])
                if b0 < _NBUF - _LOOK:
                    start_gather(c + _LOOK, fb)
                else:
                    @pl.when(i < niter - 1)
                    def _():
                        start_gather(c + _LOOK, fb)
                add_pos(rows[b0], c)
                store_chunk(rows[b0], c, ssem[b0])
            return carry

        lax.fori_loop(0, niter, loop_body, 0)
        for c in range(n_chunks - (_NBUF - _LOOK), n_chunks):
            wait_store(rows[c % _NBUF], ssem[c % _NBUF])

    return k


def kernel(token_ids, word_embeddings, positional_embeddings):
    B, T = token_ids.shape
    V, D = word_embeddings.shape
    k = _build(B, T, V, D)
    info = plsc.get_sparse_core_info()
    NW = info.num_cores * info.num_subcores
    P = T // NW
    G = _CHUNK // P
    n_chunks = B // G
    tok = (token_ids.astype(jnp.int32)
           .reshape(n_chunks, G, NW, P)
           .transpose(2, 0, 1, 3)
           .reshape(NW, n_chunks, _CHUNK))
    out = k(tok, word_embeddings, positional_embeddings)
    return out.reshape(B, T, D)


# speedup vs baseline: 1.0136x; 1.0136x over previous
import functools

import jax
import jax.numpy as jnp
from jax import lax
from jax.experimental import pallas as pl
from jax.experimental.pallas import tpu as pltpu
from jax.experimental.pallas import tpu_sc as plsc

_LANES = 16
_CHUNK = 128
_NBUF = 4
_LOOK = 3


@functools.cache
def _build(B, T, V, D):
    info = plsc.get_sparse_core_info()
    NC, NS = info.num_cores, info.num_subcores
    NW = NC * NS
    FLAT = B * T
    assert T % NW == 0 and D % _LANES == 0
    P = T // NW
    G = _CHUNK // P
    assert B % G == 0
    n_chunks = B // G
    assert n_chunks % _NBUF == 0 and n_chunks >= 2 * _NBUF
    mesh = plsc.VectorSubcoreMesh(core_axis_name="c", subcore_axis_name="s")

    @functools.partial(
        pl.kernel,
        mesh=mesh,
        out_type=jax.ShapeDtypeStruct((FLAT, D), jnp.float32),
        scratch_types=(
            [
                pltpu.VMEM((n_chunks, _CHUNK), jnp.int32),
                pltpu.VMEM((P, D), jnp.float32),
            ]
            + [pltpu.VMEM((_CHUNK, D), jnp.float32) for _ in range(_NBUF)]
            + [pltpu.SemaphoreType.DMA for _ in range(2 * _NBUF)]
        ),
    )
    def k(tok_hbm, table_hbm, pos_hbm, out_hbm, idx_v, pos_v, *bufs_and_sems):
        rows = bufs_and_sems[:_NBUF]
        gsem = bufs_and_sems[_NBUF:2 * _NBUF]
        ssem = bufs_and_sems[2 * _NBUF:]
        wid = lax.axis_index("s") * NC + lax.axis_index("c")
        pltpu.sync_copy(tok_hbm.at[wid], idx_v)
        pltpu.sync_copy(pos_hbm.at[pl.ds(wid * P, P)], pos_v)

        def start_gather(c, b):
            pltpu.async_copy(table_hbm.at[idx_v.at[c]], rows[b], gsem[b])

        def wait_gather(c, b):
            pltpu.make_async_copy(
                table_hbm.at[idx_v.at[c]], rows[b], gsem[b]).wait()

        def add_pos(rows_v, c):
            @plsc.parallel_loop(0, P, 1, unroll=2)
            def add_p(p):
                for j in range(D // _LANES):
                    sl = pl.ds(j * _LANES, _LANES)
                    pv = pos_v[p, sl]
                    for g in range(G):
                        r = g * P + p
                        rows_v[r, sl] = rows_v[r, sl] + pv

        def store_chunk(rows_v, c, sem):
            for g in range(G):
                row0 = (c * G + g) * T + wid * P
                pltpu.async_copy(
                    rows_v.at[pl.ds(g * P, P)],
                    out_hbm.at[pl.ds(row0, P)], sem)

        def wait_store(rows_v, sem):
            pltpu.make_async_copy(
                rows_v, out_hbm.at[pl.ds(0, _CHUNK)], sem).wait()

        niter = n_chunks // _NBUF
        for c0 in range(_LOOK):
            start_gather(c0, c0)

        def loop_body(i, carry):
            for b0 in range(_NBUF):
                c = _NBUF * i + b0
                wait_gather(c, b0)
                fb = (b0 + _LOOK) % _NBUF
                if b0 >= _NBUF - _LOOK:
                    wait_store(rows[fb], ssem[fb])
                else:
                    @pl.when(i > 0)
                    def _():
                        wait_store(rows[fb], ssem[fb])
                if b0 < _NBUF - _LOOK:
                    start_gather(c + _LOOK, fb)
                else:
                    @pl.when(i < niter - 1)
                    def _():
                        start_gather(c + _LOOK, fb)
                add_pos(rows[b0], c)
                store_chunk(rows[b0], c, ssem[b0])
            return carry

        lax.fori_loop(0, niter, loop_body, 0)
        for c in range(n_chunks - (_NBUF - _LOOK), n_chunks):
            wait_store(rows[c % _NBUF], ssem[c % _NBUF])

    return k


def kernel(token_ids, word_embeddings, positional_embeddings):
    B, T = token_ids.shape
    V, D = word_embeddings.shape
    k = _build(B, T, V, D)
    info = plsc.get_sparse_core_info()
    NW = info.num_cores * info.num_subcores
    P = T // NW
    G = _CHUNK // P
    n_chunks = B // G
    tok = (token_ids.astype(jnp.int32)
           .reshape(n_chunks, G, NW, P)
           .transpose(2, 0, 1, 3)
           .reshape(NW, n_chunks, _CHUNK))
    out = k(tok, word_embeddings, positional_embeddings)
    return out.reshape(B, T, D)


# speedup vs baseline: 1.0199x; 1.0063x over previous
import functools

import jax
import jax.numpy as jnp
from jax import lax
from jax.experimental import pallas as pl
from jax.experimental.pallas import tpu as pltpu
from jax.experimental.pallas import tpu_sc as plsc

_LANES = 16
_CHUNK = 128
_NBUF = 4
_LOOK = 2


@functools.cache
def _build(B, T, V, D):
    info = plsc.get_sparse_core_info()
    NC, NS = info.num_cores, info.num_subcores
    NW = NC * NS
    FLAT = B * T
    assert T % NW == 0 and D % _LANES == 0
    P = T // NW
    G = _CHUNK // P
    assert B % G == 0
    n_chunks = B // G
    assert n_chunks % _NBUF == 0 and n_chunks >= 2 * _NBUF
    mesh = plsc.VectorSubcoreMesh(core_axis_name="c", subcore_axis_name="s")

    @functools.partial(
        pl.kernel,
        mesh=mesh,
        out_type=jax.ShapeDtypeStruct((FLAT, D), jnp.float32),
        scratch_types=(
            [
                pltpu.VMEM((n_chunks, _CHUNK), jnp.int32),
                pltpu.VMEM((P, D), jnp.float32),
            ]
            + [pltpu.VMEM((_CHUNK, D), jnp.float32) for _ in range(_NBUF)]
            + [pltpu.SemaphoreType.DMA for _ in range(2 * _NBUF)]
        ),
    )
    def k(tok_hbm, table_hbm, pos_hbm, out_hbm, idx_v, pos_v, *bufs_and_sems):
        rows = bufs_and_sems[:_NBUF]
        gsem = bufs_and_sems[_NBUF:2 * _NBUF]
        ssem = bufs_and_sems[2 * _NBUF:]
        wid = lax.axis_index("s") * NC + lax.axis_index("c")
        pltpu.sync_copy(tok_hbm.at[wid], idx_v)
        pltpu.sync_copy(pos_hbm.at[pl.ds(wid * P, P)], pos_v)

        def start_gather(c, b):
            pltpu.async_copy(table_hbm.at[idx_v.at[c]], rows[b], gsem[b])

        def wait_gather(c, b):
            pltpu.make_async_copy(
                table_hbm.at[idx_v.at[c]], rows[b], gsem[b]).wait()

        def add_pos(rows_v, c):
            @plsc.parallel_loop(0, P, 1, unroll=2)
            def add_p(p):
                for j in range(D // _LANES):
                    sl = pl.ds(j * _LANES, _LANES)
                    pv = pos_v[p, sl]
                    for g in range(G):
                        r = g * P + p
                        rows_v[r, sl] = rows_v[r, sl] + pv

        def store_chunk(rows_v, c, sem):
            for g in range(G):
                row0 = (c * G + g) * T + wid * P
                pltpu.async_copy(
                    rows_v.at[pl.ds(g * P, P)],
                    out_hbm.at[pl.ds(row0, P)], sem)

        def wait_store(rows_v, sem):
            pltpu.make_async_copy(
                rows_v, out_hbm.at[pl.ds(0, _CHUNK)], sem).wait()

        niter = n_chunks // _NBUF
        for c0 in range(_LOOK):
            start_gather(c0, c0)

        def loop_body(i, carry):
            for b0 in range(_NBUF):
                c = _NBUF * i + b0
                wait_gather(c, b0)
                fb = (b0 + _LOOK) % _NBUF
                if b0 >= _NBUF - _LOOK:
                    wait_store(rows[fb], ssem[fb])
                else:
                    @pl.when(i > 0)
                    def _():
                        wait_store(rows[fb], ssem[fb])
                if b0 < _NBUF - _LOOK:
                    start_gather(c + _LOOK, fb)
                else:
                    @pl.when(i < niter - 1)
                    def _():
                        start_gather(c + _LOOK, fb)
                add_pos(rows[b0], c)
                store_chunk(rows[b0], c, ssem[b0])
            return carry

        lax.fori_loop(0, niter, loop_body, 0)
        for c in range(n_chunks - (_NBUF - _LOOK), n_chunks):
            wait_store(rows[c % _NBUF], ssem[c % _NBUF])

    return k


def kernel(token_ids, word_embeddings, positional_embeddings):
    B, T = token_ids.shape
    V, D = word_embeddings.shape
    k = _build(B, T, V, D)
    info = plsc.get_sparse_core_info()
    NW = info.num_cores * info.num_subcores
    P = T // NW
    G = _CHUNK // P
    n_chunks = B // G
    tok = (token_ids.astype(jnp.int32)
           .reshape(n_chunks, G, NW, P)
           .transpose(2, 0, 1, 3)
           .reshape(NW, n_chunks, _CHUNK))
    out = k(tok, word_embeddings, positional_embeddings)
    return out.reshape(B, T, D)


# speedup vs baseline: 1.0562x; 1.0355x over previous
import functools

import jax
import jax.numpy as jnp
from jax import lax
from jax.experimental import pallas as pl
from jax.experimental.pallas import tpu as pltpu
from jax.experimental.pallas import tpu_sc as plsc

_LANES = 16
_CHUNK = 128
_NBUF = 4
_LOOK = 2


@functools.cache
def _build(B, T, V, D):
    info = plsc.get_sparse_core_info()
    NC, NS = info.num_cores, info.num_subcores
    NW = NC * NS
    FLAT = B * T
    assert T % NW == 0 and D % _LANES == 0
    P = T // NW
    G = _CHUNK // P
    assert B % G == 0
    n_chunks = B // G
    assert n_chunks % _NBUF == 0 and n_chunks >= 2 * _NBUF
    mesh = plsc.VectorSubcoreMesh(core_axis_name="c", subcore_axis_name="s")

    @functools.partial(
        pl.kernel,
        mesh=mesh,
        out_type=jax.ShapeDtypeStruct((FLAT, D), jnp.float32),
        scratch_types=(
            [
                pltpu.VMEM((_NBUF, _CHUNK), jnp.int32),
                pltpu.VMEM((2, 1, G, 1, P), jnp.int32),
                pltpu.VMEM((P, D), jnp.float32),
            ]
            + [pltpu.VMEM((_CHUNK, D), jnp.float32) for _ in range(_NBUF)]
            + [pltpu.SemaphoreType.DMA for _ in range(2 * _NBUF)]
            + [pltpu.SemaphoreType.DMA, pltpu.SemaphoreType.DMA]
        ),
    )
    def k(tok_hbm, table_hbm, pos_hbm, out_hbm, idx_v, stg_v, pos_v,
          *bufs_and_sems):
        rows = bufs_and_sems[:_NBUF]
        gsem = bufs_and_sems[_NBUF:2 * _NBUF]
        ssem = bufs_and_sems[2 * _NBUF:3 * _NBUF]
        tsem = bufs_and_sems[3 * _NBUF:]
        wid = lax.axis_index("s") * NC + lax.axis_index("c")
        pltpu.sync_copy(pos_hbm.at[pl.ds(wid * P, P)], pos_v)

        def tok_src(c):
            return tok_hbm.at[pl.ds(c, 1), :, pl.ds(wid, 1), :]

        def start_tok(c, s):
            pltpu.async_copy(tok_src(c), stg_v.at[s], tsem[s])

        def wait_tok(c, s):
            pltpu.make_async_copy(tok_src(c), stg_v.at[s], tsem[s]).wait()

        def relay_tok(s, ib):
            for g in range(G):
                idx_v[ib, pl.ds(g * P, P)] = stg_v[s, 0, g, 0, :]

        def start_gather(c, b, ib):
            pltpu.async_copy(table_hbm.at[idx_v.at[ib]], rows[b], gsem[b])

        def wait_gather(c, b, ib):
            pltpu.make_async_copy(
                table_hbm.at[idx_v.at[ib]], rows[b], gsem[b]).wait()

        def add_pos(rows_v, c):
            @plsc.parallel_loop(0, P, 1, unroll=2)
            def add_p(p):
                for j in range(D // _LANES):
                    sl = pl.ds(j * _LANES, _LANES)
                    pv = pos_v[p, sl]
                    for g in range(G):
                        r = g * P + p
                        rows_v[r, sl] = rows_v[r, sl] + pv

        def store_chunk(rows_v, c, sem):
            for g in range(G):
                row0 = (c * G + g) * T + wid * P
                pltpu.async_copy(
                    rows_v.at[pl.ds(g * P, P)],
                    out_hbm.at[pl.ds(row0, P)], sem)

        def wait_store(rows_v, sem):
            pltpu.make_async_copy(
                rows_v, out_hbm.at[pl.ds(0, _CHUNK)], sem).wait()

        niter = n_chunks // _NBUF
        for c0 in range(_LOOK):
            start_tok(c0, c0 % 2)
            wait_tok(c0, c0 % 2)
            relay_tok(c0 % 2, c0 % _NBUF)
            start_gather(c0, c0, c0 % _NBUF)
        start_tok(_LOOK, _LOOK % 2)

        def loop_body(i, carry):
            for b0 in range(_NBUF):
                c = _NBUF * i + b0
                wait_gather(c, b0, b0)
                fb = (b0 + _LOOK) % _NBUF
                if b0 >= _NBUF - _LOOK:
                    wait_store(rows[fb], ssem[fb])
                else:
                    @pl.when(i > 0)
                    def _():
                        wait_store(rows[fb], ssem[fb])

                def lookahead(i):
                    cg = _NBUF * i + b0 + _LOOK
                    wait_tok(cg, (b0 + _LOOK) % 2)
                    relay_tok((b0 + _LOOK) % 2, fb)
                    start_gather(cg, fb, fb)

                def prefetch(i):
                    cn = _NBUF * i + b0 + _LOOK + 1
                    start_tok(cn, (b0 + _LOOK + 1) % 2)

                if b0 < _NBUF - _LOOK:
                    lookahead(i)
                else:
                    @pl.when(i < niter - 1)
                    def _():
                        lookahead(i)
                if b0 < _NBUF - _LOOK - 1:
                    prefetch(i)
                else:
                    @pl.when(i < niter - 1)
                    def _():
                        prefetch(i)
                add_pos(rows[b0], c)
                store_chunk(rows[b0], c, ssem[b0])
            return carry

        lax.fori_loop(0, niter, loop_body, 0)
        for c in range(n_chunks - (_NBUF - _LOOK), n_chunks):
            wait_store(rows[c % _NBUF], ssem[c % _NBUF])

    return k


def kernel(token_ids, word_embeddings, positional_embeddings):
    B, T = token_ids.shape
    V, D = word_embeddings.shape
    k = _build(B, T, V, D)
    info = plsc.get_sparse_core_info()
    NW = info.num_cores * info.num_subcores
    P = T // NW
    G = _CHUNK // P
    n_chunks = B // G
    tok = token_ids.astype(jnp.int32).reshape(n_chunks, G, NW, P)
    out = k(tok, word_embeddings, positional_embeddings)
    return out.reshape(B, T, D)


# speedup vs baseline: 1.0673x; 1.0105x over previous
import functools

import jax
import jax.numpy as jnp
from jax import lax
from jax.experimental import pallas as pl
from jax.experimental.pallas import tpu as pltpu
from jax.experimental.pallas import tpu_sc as plsc

_LANES = 16
_CHUNK = 128
_NBUF = 4
_LOOK = 2


@functools.cache
def _build(B, T, V, D):
    info = plsc.get_sparse_core_info()
    NC, NS = info.num_cores, info.num_subcores
    NW = NC * NS
    FLAT = B * T
    assert T % NW == 0 and D % _LANES == 0
    P = T // NW
    G = _CHUNK // P
    assert B % G == 0
    n_chunks = B // G
    assert n_chunks % _NBUF == 0 and n_chunks >= 2 * _NBUF
    mesh = plsc.VectorSubcoreMesh(core_axis_name="c", subcore_axis_name="s")

    @functools.partial(
        pl.kernel,
        mesh=mesh,
        out_type=jax.ShapeDtypeStruct((FLAT, D), jnp.float32),
        scratch_types=(
            [
                pltpu.VMEM((_NBUF, _CHUNK), jnp.int32),
                pltpu.VMEM((4, 1, G, 1, P), jnp.int32),
                pltpu.VMEM((P, D), jnp.float32),
            ]
            + [pltpu.VMEM((_CHUNK, D), jnp.float32) for _ in range(_NBUF)]
            + [pltpu.SemaphoreType.DMA for _ in range(2 * _NBUF)]
            + [pltpu.SemaphoreType.DMA for _ in range(4)]
        ),
    )
    def k(tok_hbm, table_hbm, pos_hbm, out_hbm, idx_v, stg_v, pos_v,
          *bufs_and_sems):
        rows = bufs_and_sems[:_NBUF]
        gsem = bufs_and_sems[_NBUF:2 * _NBUF]
        ssem = bufs_and_sems[2 * _NBUF:3 * _NBUF]
        tsem = bufs_and_sems[3 * _NBUF:]
        wid = lax.axis_index("s") * NC + lax.axis_index("c")
        pltpu.sync_copy(pos_hbm.at[pl.ds(wid * P, P)], pos_v)

        def tok_src(c):
            return tok_hbm.at[pl.ds(c, 1), :, pl.ds(wid, 1), :]

        def start_tok(c, s):
            pltpu.async_copy(tok_src(c), stg_v.at[s], tsem[s])

        def wait_tok(c, s):
            pltpu.make_async_copy(tok_src(c), stg_v.at[s], tsem[s]).wait()

        def relay_tok(s, ib):
            for g in range(G):
                idx_v[ib, pl.ds(g * P, P)] = stg_v[s, 0, g, 0, :]

        def start_gather(c, b, ib):
            pltpu.async_copy(table_hbm.at[idx_v.at[ib]], rows[b], gsem[b])

        def wait_gather(c, b, ib):
            pltpu.make_async_copy(
                table_hbm.at[idx_v.at[ib]], rows[b], gsem[b]).wait()

        def add_pos(rows_v, c):
            @plsc.parallel_loop(0, P, 1, unroll=2)
            def add_p(p):
                for j in range(D // _LANES):
                    sl = pl.ds(j * _LANES, _LANES)
                    pv = pos_v[p, sl]
                    for g in range(G):
                        r = g * P + p
                        rows_v[r, sl] = rows_v[r, sl] + pv

        def store_chunk(rows_v, c, sem):
            for g in range(G):
                row0 = (c * G + g) * T + wid * P
                pltpu.async_copy(
                    rows_v.at[pl.ds(g * P, P)],
                    out_hbm.at[pl.ds(row0, P)], sem)

        def wait_store(rows_v, sem):
            pltpu.make_async_copy(
                rows_v, out_hbm.at[pl.ds(0, _CHUNK)], sem).wait()

        niter = n_chunks // _NBUF
        for c0 in range(_LOOK):
            start_tok(c0, c0 % 4)
            wait_tok(c0, c0 % 4)
            relay_tok(c0 % 4, c0 % _NBUF)
            start_gather(c0, c0, c0 % _NBUF)
        start_tok(_LOOK, _LOOK % 4)
        start_tok(_LOOK + 1, (_LOOK + 1) % 4)

        def loop_body(i, carry):
            for b0 in range(_NBUF):
                c = _NBUF * i + b0
                wait_gather(c, b0, b0)
                fb = (b0 + _LOOK) % _NBUF
                if b0 >= _NBUF - _LOOK:
                    wait_store(rows[fb], ssem[fb])
                else:
                    @pl.when(i > 0)
                    def _():
                        wait_store(rows[fb], ssem[fb])

                def lookahead(i):
                    cg = _NBUF * i + b0 + _LOOK
                    wait_tok(cg, (b0 + _LOOK) % 4)
                    relay_tok((b0 + _LOOK) % 4, fb)
                    start_gather(cg, fb, fb)

                def prefetch(i):
                    cn = _NBUF * i + b0 + _LOOK + 2
                    start_tok(cn, (b0 + _LOOK + 2) % 4)

                if b0 < _NBUF - _LOOK:
                    lookahead(i)
                else:
                    @pl.when(i < niter - 1)
                    def _():
                        lookahead(i)

                @pl.when(i < niter - 1)
                def _():
                    prefetch(i)
                add_pos(rows[b0], c)
                store_chunk(rows[b0], c, ssem[b0])
            return carry

        lax.fori_loop(0, niter, loop_body, 0)
        for c in range(n_chunks - (_NBUF - _LOOK), n_chunks):
            wait_store(rows[c % _NBUF], ssem[c % _NBUF])

    return k


def kernel(token_ids, word_embeddings, positional_embeddings):
    B, T = token_ids.shape
    V, D = word_embeddings.shape
    k = _build(B, T, V, D)
    info = plsc.get_sparse_core_info()
    NW = info.num_cores * info.num_subcores
    P = T // NW
    G = _CHUNK // P
    n_chunks = B // G
    tok = token_ids.astype(jnp.int32).reshape(n_chunks, G, NW, P)
    out = k(tok, word_embeddings, positional_embeddings)
    return out.reshape(B, T, D)


# speedup vs baseline: 1.0861x; 1.0176x over previous
import functools

import jax
import jax.numpy as jnp
from jax import lax
from jax.experimental import pallas as pl
from jax.experimental.pallas import tpu as pltpu
from jax.experimental.pallas import tpu_sc as plsc

_LANES = 16
_CHUNK = 128
_NBUF = 4
_LOOK = 2
_S = 2


@functools.cache
def _build(B, T, V, D):
    info = plsc.get_sparse_core_info()
    NC, NS = info.num_cores, info.num_subcores
    NW = NC * NS
    FLAT = B * T
    TS = NW // _S
    assert T % TS == 0 and D % _LANES == 0
    P = T // TS
    G = _CHUNK // P
    BG = B // _S
    assert BG % G == 0
    n_chunks = BG // G
    assert n_chunks % _NBUF == 0 and n_chunks >= 2 * _NBUF
    mesh = plsc.VectorSubcoreMesh(core_axis_name="c", subcore_axis_name="s")

    @functools.partial(
        pl.kernel,
        mesh=mesh,
        out_type=jax.ShapeDtypeStruct((FLAT, D), jnp.float32),
        scratch_types=(
            [
                pltpu.VMEM((_NBUF, _CHUNK), jnp.int32),
                pltpu.VMEM((2, 1, 1, G, 1, P), jnp.int32),
                pltpu.VMEM((P, D), jnp.float32),
            ]
            + [pltpu.VMEM((_CHUNK, D), jnp.float32) for _ in range(_NBUF)]
            + [pltpu.SemaphoreType.DMA for _ in range(2 * _NBUF)]
            + [pltpu.SemaphoreType.DMA, pltpu.SemaphoreType.DMA]
        ),
    )
    def k(tok_hbm, table_hbm, pos_hbm, out_hbm, idx_v, stg_v, pos_v,
          *bufs_and_sems):
        rows = bufs_and_sems[:_NBUF]
        gsem = bufs_and_sems[_NBUF:2 * _NBUF]
        ssem = bufs_and_sems[2 * _NBUF:3 * _NBUF]
        tsem = bufs_and_sems[3 * _NBUF:]
        wid = lax.axis_index("s") * NC + lax.axis_index("c")
        bh = lax.div(wid, TS)
        ts = lax.rem(wid, TS)
        pltpu.sync_copy(pos_hbm.at[pl.ds(ts * P, P)], pos_v)

        def tok_src(c):
            return tok_hbm.at[pl.ds(bh, 1), pl.ds(c, 1), :, pl.ds(ts, 1), :]

        def start_tok(c, s):
            pltpu.async_copy(tok_src(c), stg_v.at[s], tsem[s])

        def wait_tok(c, s):
            pltpu.make_async_copy(tok_src(c), stg_v.at[s], tsem[s]).wait()

        def relay_tok(s, ib):
            for g in range(G):
                for q in range(P // _LANES):
                    idx_v[ib, pl.ds(g * P + q * _LANES, _LANES)] = (
                        stg_v[s, 0, 0, g, 0, pl.ds(q * _LANES, _LANES)])

        def start_gather(c, b, ib):
            pltpu.async_copy(table_hbm.at[idx_v.at[ib]], rows[b], gsem[b])

        def wait_gather(c, b, ib):
            pltpu.make_async_copy(
                table_hbm.at[idx_v.at[ib]], rows[b], gsem[b]).wait()

        def add_pos(rows_v, c):
            @plsc.parallel_loop(0, P, 1, unroll=2)
            def add_p(p):
                for j in range(D // _LANES):
                    sl = pl.ds(j * _LANES, _LANES)
                    pv = pos_v[p, sl]
                    for g in range(G):
                        r = g * P + p
                        rows_v[r, sl] = rows_v[r, sl] + pv

        def store_chunk(rows_v, c, sem):
            for g in range(G):
                row0 = (bh * BG + c * G + g) * T + ts * P
                pltpu.async_copy(
                    rows_v.at[pl.ds(g * P, P)],
                    out_hbm.at[pl.ds(row0, P)], sem)

        def wait_store(rows_v, sem):
            pltpu.make_async_copy(
                rows_v, out_hbm.at[pl.ds(0, _CHUNK)], sem).wait()

        niter = n_chunks // _NBUF
        for c0 in range(_LOOK):
            start_tok(c0, c0 % 2)
            wait_tok(c0, c0 % 2)
            relay_tok(c0 % 2, c0 % _NBUF)
            start_gather(c0, c0, c0 % _NBUF)
        start_tok(_LOOK, _LOOK % 2)

        def loop_body(i, carry):
            for b0 in range(_NBUF):
                c = _NBUF * i + b0
                wait_gather(c, b0, b0)
                fb = (b0 + _LOOK) % _NBUF
                if b0 >= _NBUF - _LOOK:
                    wait_store(rows[fb], ssem[fb])
                else:
                    @pl.when(i > 0)
                    def _():
                        wait_store(rows[fb], ssem[fb])

                def lookahead(i):
                    cg = _NBUF * i + b0 + _LOOK
                    wait_tok(cg, (b0 + _LOOK) % 2)
                    relay_tok((b0 + _LOOK) % 2, fb)
                    start_gather(cg, fb, fb)

                def prefetch(i):
                    cn = _NBUF * i + b0 + _LOOK + 1
                    start_tok(cn, (b0 + _LOOK + 1) % 2)

                if b0 < _NBUF - _LOOK:
                    lookahead(i)
                else:
                    @pl.when(i < niter - 1)
                    def _():
                        lookahead(i)
                if b0 < _NBUF - _LOOK - 1:
                    prefetch(i)
                else:
                    @pl.when(i < niter - 1)
                    def _():
                        prefetch(i)
                add_pos(rows[b0], c)
                store_chunk(rows[b0], c, ssem[b0])
            return carry

        lax.fori_loop(0, niter, loop_body, 0)
        for c in range(n_chunks - (_NBUF - _LOOK), n_chunks):
            wait_store(rows[c % _NBUF], ssem[c % _NBUF])

    return k


def kernel(token_ids, word_embeddings, positional_embeddings):
    B, T = token_ids.shape
    V, D = word_embeddings.shape
    k = _build(B, T, V, D)
    info = plsc.get_sparse_core_info()
    NW = info.num_cores * info.num_subcores
    TS = NW // _S
    P = T // TS
    G = _CHUNK // P
    n_chunks = (B // _S) // G
    tok = token_ids.astype(jnp.int32).reshape(_S, n_chunks, G, TS, P)
    out = k(tok, word_embeddings, positional_embeddings)
    return out.reshape(B, T, D)


# speedup vs baseline: 1.1073x; 1.0196x over previous
import functools

import jax
import jax.numpy as jnp
from jax import lax
from jax.experimental import pallas as pl
from jax.experimental.pallas import tpu as pltpu
from jax.experimental.pallas import tpu_sc as plsc

_LANES = 16
_CHUNK = 128
_NBUF = 4
_LOOK = 2
_S = 4


@functools.cache
def _build(B, T, V, D):
    info = plsc.get_sparse_core_info()
    NC, NS = info.num_cores, info.num_subcores
    NW = NC * NS
    FLAT = B * T
    TS = NW // _S
    assert T % TS == 0 and D % _LANES == 0
    P = T // TS
    G = _CHUNK // P
    BG = B // _S
    assert BG % G == 0
    n_chunks = BG // G
    assert n_chunks % _NBUF == 0 and n_chunks >= 2 * _NBUF
    mesh = plsc.VectorSubcoreMesh(core_axis_name="c", subcore_axis_name="s")

    @functools.partial(
        pl.kernel,
        mesh=mesh,
        out_type=jax.ShapeDtypeStruct((FLAT, D), jnp.float32),
        scratch_types=(
            [
                pltpu.VMEM((_NBUF, _CHUNK), jnp.int32),
                pltpu.VMEM((2, 1, 1, G, 1, P), jnp.int32),
                pltpu.VMEM((P, D), jnp.float32),
            ]
            + [pltpu.VMEM((_CHUNK, D), jnp.float32) for _ in range(_NBUF)]
            + [pltpu.SemaphoreType.DMA for _ in range(2 * _NBUF)]
            + [pltpu.SemaphoreType.DMA, pltpu.SemaphoreType.DMA]
        ),
    )
    def k(tok_hbm, table_hbm, pos_hbm, out_hbm, idx_v, stg_v, pos_v,
          *bufs_and_sems):
        rows = bufs_and_sems[:_NBUF]
        gsem = bufs_and_sems[_NBUF:2 * _NBUF]
        ssem = bufs_and_sems[2 * _NBUF:3 * _NBUF]
        tsem = bufs_and_sems[3 * _NBUF:]
        wid = lax.axis_index("s") * NC + lax.axis_index("c")
        bh = lax.div(wid, TS)
        ts = lax.rem(wid, TS)
        pltpu.sync_copy(pos_hbm.at[pl.ds(ts * P, P)], pos_v)

        def tok_src(c):
            return tok_hbm.at[pl.ds(bh, 1), pl.ds(c, 1), :, pl.ds(ts, 1), :]

        def start_tok(c, s):
            pltpu.async_copy(tok_src(c), stg_v.at[s], tsem[s])

        def wait_tok(c, s):
            pltpu.make_async_copy(tok_src(c), stg_v.at[s], tsem[s]).wait()

        def relay_tok(s, ib):
            for g in range(G):
                for q in range(P // _LANES):
                    idx_v[ib, pl.ds(g * P + q * _LANES, _LANES)] = (
                        stg_v[s, 0, 0, g, 0, pl.ds(q * _LANES, _LANES)])

        def start_gather(c, b, ib):
            pltpu.async_copy(table_hbm.at[idx_v.at[ib]], rows[b], gsem[b])

        def wait_gather(c, b, ib):
            pltpu.make_async_copy(
                table_hbm.at[idx_v.at[ib]], rows[b], gsem[b]).wait()

        def add_pos(rows_v, c):
            @plsc.parallel_loop(0, P, 1, unroll=2)
            def add_p(p):
                for j in range(D // _LANES):
                    sl = pl.ds(j * _LANES, _LANES)
                    pv = pos_v[p, sl]
                    for g in range(G):
                        r = g * P + p
                        rows_v[r, sl] = rows_v[r, sl] + pv

        def store_chunk(rows_v, c, sem):
            for g in range(G):
                row0 = (bh * BG + c * G + g) * T + ts * P
                pltpu.async_copy(
                    rows_v.at[pl.ds(g * P, P)],
                    out_hbm.at[pl.ds(row0, P)], sem)

        def wait_store(rows_v, sem):
            pltpu.make_async_copy(
                rows_v, out_hbm.at[pl.ds(0, _CHUNK)], sem).wait()

        niter = n_chunks // _NBUF
        for c0 in range(_LOOK):
            start_tok(c0, c0 % 2)
            wait_tok(c0, c0 % 2)
            relay_tok(c0 % 2, c0 % _NBUF)
            start_gather(c0, c0, c0 % _NBUF)
        start_tok(_LOOK, _LOOK % 2)

        def loop_body(i, carry):
            for b0 in range(_NBUF):
                c = _NBUF * i + b0
                wait_gather(c, b0, b0)
                fb = (b0 + _LOOK) % _NBUF
                if b0 >= _NBUF - _LOOK:
                    wait_store(rows[fb], ssem[fb])
                else:
                    @pl.when(i > 0)
                    def _():
                        wait_store(rows[fb], ssem[fb])

                def lookahead(i):
                    cg = _NBUF * i + b0 + _LOOK
                    wait_tok(cg, (b0 + _LOOK) % 2)
                    relay_tok((b0 + _LOOK) % 2, fb)
                    start_gather(cg, fb, fb)

                def prefetch(i):
                    cn = _NBUF * i + b0 + _LOOK + 1
                    start_tok(cn, (b0 + _LOOK + 1) % 2)

                if b0 < _NBUF - _LOOK:
                    lookahead(i)
                else:
                    @pl.when(i < niter - 1)
                    def _():
                        lookahead(i)
                if b0 < _NBUF - _LOOK - 1:
                    prefetch(i)
                else:
                    @pl.when(i < niter - 1)
                    def _():
                        prefetch(i)
                add_pos(rows[b0], c)
                store_chunk(rows[b0], c, ssem[b0])
            return carry

        lax.fori_loop(0, niter, loop_body, 0)
        for c in range(n_chunks - (_NBUF - _LOOK), n_chunks):
            wait_store(rows[c % _NBUF], ssem[c % _NBUF])

    return k


def kernel(token_ids, word_embeddings, positional_embeddings):
    B, T = token_ids.shape
    V, D = word_embeddings.shape
    k = _build(B, T, V, D)
    info = plsc.get_sparse_core_info()
    NW = info.num_cores * info.num_subcores
    TS = NW // _S
    P = T // TS
    G = _CHUNK // P
    n_chunks = (B // _S) // G
    tok = token_ids.astype(jnp.int32).reshape(_S, n_chunks, G, TS, P)
    out = k(tok, word_embeddings, positional_embeddings)
    return out.reshape(B, T, D)
